# baseline (device time: 47905 ns/iter reference)
import jax
import jax.numpy as jnp
from jax import lax
from jax.experimental import pallas as pl
from jax.experimental.pallas import tpu as pltpu

N_DEV = 4
N_TOK = 512
D_IN = 256
D_OUT = 512
N_EXP = 8
EXP_PER_DEV = N_EXP // N_DEV


def kernel(x, router_W, route_idx, expert_W):
    def body(x_ref, rw_ref, idx_ref, ew_ref, out_ref, comm_ref, send_sems, recv_sems):
        my = lax.axis_index("i")
        left = (my - 1) % N_DEV
        right = (my + 1) % N_DEV

        barrier_sem = pltpu.get_barrier_semaphore()
        for nbr in [left, right]:
            pl.semaphore_signal(
                barrier_sem, inc=1,
                device_id=(nbr,), device_id_type=pl.DeviceIdType.MESH,
            )
        pl.semaphore_wait(barrier_sem, 2)

        xv = x_ref[:, :]
        scores = jnp.dot(xv, rw_ref[:, :], preferred_element_type=jnp.float32)
        s_max = jnp.max(scores, axis=-1, keepdims=True)
        ex = jnp.exp(scores - s_max)
        probs = ex / jnp.sum(ex, axis=-1, keepdims=True)

        idx0 = idx_ref[:, 0:1]
        idx1 = idx_ref[:, 1:2]
        e_ids = lax.broadcasted_iota(jnp.int32, (N_TOK, N_EXP), 1)
        g0 = jnp.sum(probs * (e_ids == idx0), axis=1, keepdims=True)
        g1 = jnp.sum(probs * (e_ids == idx1), axis=1, keepdims=True)
        gs = g0 + g1

        xb = xv.astype(jnp.bfloat16)
        partial = jnp.zeros((N_TOK, D_OUT), jnp.float32)
        for k in range(EXP_PER_DEV):
            ge = EXP_PER_DEV * my + k
            sel = (idx0 == ge) | (idx1 == ge)
            p_e = jnp.sum(probs * (e_ids == ge), axis=1, keepdims=True)
            gate = jnp.where(sel, p_e / gs, 0.0)
            y = jnp.dot(xb, ew_ref[k].astype(jnp.bfloat16),
                        preferred_element_type=jnp.float32)
            partial = partial + gate * y

        comm_ref[0, :, :] = partial
        acc = partial
        for h in range(N_DEV - 1):
            rdma = pltpu.make_async_remote_copy(
                src_ref=comm_ref.at[h],
                dst_ref=comm_ref.at[h + 1],
                send_sem=send_sems.at[h],
                recv_sem=recv_sems.at[h],
                device_id=(right,),
                device_id_type=pl.DeviceIdType.MESH,
            )
            rdma.start()
            rdma.wait()
            acc = acc + comm_ref[h + 1, :, :]

        out_ref[:, :] = acc

    return pl.pallas_call(
        body,
        out_shape=jax.ShapeDtypeStruct((N_TOK, D_OUT), jnp.float32),
        in_specs=[
            pl.BlockSpec(memory_space=pltpu.VMEM),
            pl.BlockSpec(memory_space=pltpu.VMEM),
            pl.BlockSpec(memory_space=pltpu.VMEM),
            pl.BlockSpec(memory_space=pltpu.VMEM),
        ],
        out_specs=pl.BlockSpec(memory_space=pltpu.VMEM),
        scratch_shapes=[
            pltpu.VMEM((N_DEV, N_TOK, D_OUT), jnp.float32),
            pltpu.SemaphoreType.DMA((N_DEV - 1,)),
            pltpu.SemaphoreType.DMA((N_DEV - 1,)),
        ],
        compiler_params=pltpu.CompilerParams(collective_id=0),
    )(x, router_W, route_idx, expert_W)


# device time: 23251 ns/iter; 2.0603x vs baseline; 2.0603x over previous
import jax
import jax.numpy as jnp
from jax import lax
from jax.experimental import pallas as pl
from jax.experimental.pallas import tpu as pltpu

N_DEV = 4
N_TOK = 512
D_IN = 256
D_OUT = 512
N_EXP = 8
EXP_PER_DEV = N_EXP // N_DEV


def kernel(x, router_W, route_idx, expert_W):
    def body(x_ref, rw_ref, idx_ref, ew_ref, out_ref,
             send_buf, recv_buf, send_sems, recv_sems):
        my = lax.axis_index("i")
        p1 = my ^ 1
        p2 = 3 - my

        barrier_sem = pltpu.get_barrier_semaphore()
        for nbr in [p1, p2]:
            pl.semaphore_signal(
                barrier_sem, inc=1,
                device_id=(nbr,), device_id_type=pl.DeviceIdType.MESH,
            )
        pl.semaphore_wait(barrier_sem, 2)

        xv = x_ref[:, :]
        scores = jnp.dot(xv, rw_ref[:, :], preferred_element_type=jnp.float32)
        s_max = jnp.max(scores, axis=-1, keepdims=True)
        ex = jnp.exp(scores - s_max)
        probs = ex / jnp.sum(ex, axis=-1, keepdims=True)

        idx0 = idx_ref[:, 0:1]
        idx1 = idx_ref[:, 1:2]
        e_ids = lax.broadcasted_iota(jnp.int32, (N_TOK, N_EXP), 1)
        g0 = jnp.sum(probs * (e_ids == idx0), axis=1, keepdims=True)
        g1 = jnp.sum(probs * (e_ids == idx1), axis=1, keepdims=True)
        gs = g0 + g1

        xb = xv.astype(jnp.bfloat16)
        partial = jnp.zeros((N_TOK, D_OUT), jnp.float32)
        for k in range(EXP_PER_DEV):
            ge = EXP_PER_DEV * my + k
            sel = (idx0 == ge) | (idx1 == ge)
            p_e = jnp.sum(probs * (e_ids == ge), axis=1, keepdims=True)
            gate = jnp.where(sel, p_e / gs, 0.0)
            y = jnp.dot(xb, ew_ref[k].astype(jnp.bfloat16),
                        preferred_element_type=jnp.float32)
            partial = partial + gate * y

        acc = partial
        for s, p in enumerate([p1, p2]):
            send_buf[s, :, :] = acc.astype(jnp.bfloat16)
            rdma = pltpu.make_async_remote_copy(
                src_ref=send_buf.at[s],
                dst_ref=recv_buf.at[s],
                send_sem=send_sems.at[s],
                recv_sem=recv_sems.at[s],
                device_id=(p,),
                device_id_type=pl.DeviceIdType.MESH,
            )
            rdma.start()
            rdma.wait()
            acc = acc + recv_buf[s, :, :].astype(jnp.float32)

        out_ref[:, :] = acc

    return pl.pallas_call(
        body,
        out_shape=jax.ShapeDtypeStruct((N_TOK, D_OUT), jnp.float32),
        in_specs=[
            pl.BlockSpec(memory_space=pltpu.VMEM),
            pl.BlockSpec(memory_space=pltpu.VMEM),
            pl.BlockSpec(memory_space=pltpu.VMEM),
            pl.BlockSpec(memory_space=pltpu.VMEM),
        ],
        out_specs=pl.BlockSpec(memory_space=pltpu.VMEM),
        scratch_shapes=[
            pltpu.VMEM((2, N_TOK, D_OUT), jnp.bfloat16),
            pltpu.VMEM((2, N_TOK, D_OUT), jnp.bfloat16),
            pltpu.SemaphoreType.DMA((2,)),
            pltpu.SemaphoreType.DMA((2,)),
        ],
        compiler_params=pltpu.CompilerParams(collective_id=0),
    )(x, router_W, route_idx, expert_W)


# device time: 19036 ns/iter; 2.5165x vs baseline; 1.2214x over previous
import jax
import jax.numpy as jnp
from jax import lax
from jax.experimental import pallas as pl
from jax.experimental.pallas import tpu as pltpu

N_DEV = 4
N_TOK = 512
D_IN = 256
D_OUT = 512
N_EXP = 8
EXP_PER_DEV = N_EXP // N_DEV
N_CHUNK = 4


def kernel(x, router_W, route_idx, expert_W):
    def body(x_ref, rw_ref, idx_ref, ew_ref, out_ref,
             send_buf, recv_buf, send_sems, recv_sems):
        my = lax.axis_index("i")
        p1 = my ^ 1
        p2 = 3 - my

        barrier_sem = pltpu.get_barrier_semaphore()
        for nbr in [p1, p2]:
            pl.semaphore_signal(
                barrier_sem, inc=1,
                device_id=(nbr,), device_id_type=pl.DeviceIdType.MESH,
            )
        pl.semaphore_wait(barrier_sem, 2)

        xv = x_ref[:, :]
        scores = jnp.dot(xv, rw_ref[:, :], preferred_element_type=jnp.float32)
        s_max = jnp.max(scores, axis=-1, keepdims=True)
        ex = jnp.exp(scores - s_max)
        probs = ex / jnp.sum(ex, axis=-1, keepdims=True)

        idx0 = idx_ref[:, 0:1]
        idx1 = idx_ref[:, 1:2]
        e_ids = lax.broadcasted_iota(jnp.int32, (N_TOK, N_EXP), 1)
        g0 = jnp.sum(probs * (e_ids == idx0), axis=1, keepdims=True)
        g1 = jnp.sum(probs * (e_ids == idx1), axis=1, keepdims=True)
        gs = g0 + g1

        xb = xv.astype(jnp.bfloat16)
        partial = jnp.zeros((N_TOK, D_OUT), jnp.float32)
        for k in range(EXP_PER_DEV):
            ge = EXP_PER_DEV * my + k
            sel = (idx0 == ge) | (idx1 == ge)
            p_e = jnp.sum(probs * (e_ids == ge), axis=1, keepdims=True)
            gate = jnp.where(sel, p_e / gs, 0.0)
            y = jnp.dot(xb, ew_ref[k].astype(jnp.bfloat16),
                        preferred_element_type=jnp.float32)
            partial = partial + gate * y

        CK = N_TOK // N_CHUNK

        def mk(stage, c, p):
            sl = pl.ds(c * CK, CK)
            return pltpu.make_async_remote_copy(
                src_ref=send_buf.at[stage, sl],
                dst_ref=recv_buf.at[stage, sl],
                send_sem=send_sems.at[stage, c],
                recv_sem=recv_sems.at[stage, c],
                device_id=(p,),
                device_id_type=pl.DeviceIdType.MESH,
            )

        send_buf[0, :, :] = partial.astype(jnp.bfloat16)
        s1 = [mk(0, c, p1) for c in range(N_CHUNK)]
        s2 = [mk(1, c, p2) for c in range(N_CHUNK)]
        for c in range(N_CHUNK):
            s1[c].start()

        accs = []
        for c in range(N_CHUNK):
            s1[c].wait()
            sl = pl.ds(c * CK, CK)
            a = partial[c * CK:(c + 1) * CK, :] + recv_buf[0, sl, :].astype(jnp.float32)
            accs.append(a)
            send_buf[1, sl, :] = a.astype(jnp.bfloat16)
            s2[c].start()

        for c in range(N_CHUNK):
            s2[c].wait()
            sl = pl.ds(c * CK, CK)
            out_ref[sl, :] = accs[c] + recv_buf[1, sl, :].astype(jnp.float32)

    return pl.pallas_call(
        body,
        out_shape=jax.ShapeDtypeStruct((N_TOK, D_OUT), jnp.float32),
        in_specs=[
            pl.BlockSpec(memory_space=pltpu.VMEM),
            pl.BlockSpec(memory_space=pltpu.VMEM),
            pl.BlockSpec(memory_space=pltpu.VMEM),
            pl.BlockSpec(memory_space=pltpu.VMEM),
        ],
        out_specs=pl.BlockSpec(memory_space=pltpu.VMEM),
        scratch_shapes=[
            pltpu.VMEM((2, N_TOK, D_OUT), jnp.bfloat16),
            pltpu.VMEM((2, N_TOK, D_OUT), jnp.bfloat16),
            pltpu.SemaphoreType.DMA((2, N_CHUNK)),
            pltpu.SemaphoreType.DMA((2, N_CHUNK)),
        ],
        compiler_params=pltpu.CompilerParams(collective_id=0),
    )(x, router_W, route_idx, expert_W)


# device time: 15732 ns/iter; 3.0451x vs baseline; 1.2100x over previous
import jax
import jax.numpy as jnp
from jax import lax
from jax.experimental import pallas as pl
from jax.experimental.pallas import tpu as pltpu

N_DEV = 4
N_TOK = 512
D_IN = 256
D_OUT = 512
N_EXP = 8
EXP_PER_DEV = N_EXP // N_DEV
N_CHUNK = 8


def kernel(x, router_W, route_idx, expert_W):
    def body(x_ref, rw_ref, idx_ref, ew_ref, out_ref,
             send_buf, recv_buf, ew_vmem, ew_sem, send_sems, recv_sems):
        my = lax.axis_index("i")
        p1 = my ^ 1
        p2 = 3 - my

        barrier_sem = pltpu.get_barrier_semaphore()
        for nbr in [p1, p2]:
            pl.semaphore_signal(
                barrier_sem, inc=1,
                device_id=(nbr,), device_id_type=pl.DeviceIdType.MESH,
            )

        ew_cp = pltpu.make_async_copy(ew_ref, ew_vmem, ew_sem)
        ew_cp.start()

        xv = x_ref[:, :]
        scores = jnp.dot(xv, rw_ref[:, :], preferred_element_type=jnp.float32)
        s_max = jnp.max(scores, axis=-1, keepdims=True)
        ex = jnp.exp(scores - s_max)
        probs = ex / jnp.sum(ex, axis=-1, keepdims=True)

        idx0 = idx_ref[:, 0:1]
        idx1 = idx_ref[:, 1:2]
        e_ids = lax.broadcasted_iota(jnp.int32, (N_TOK, N_EXP), 1)
        g0 = jnp.sum(probs * (e_ids == idx0), axis=1, keepdims=True)
        g1 = jnp.sum(probs * (e_ids == idx1), axis=1, keepdims=True)
        gs = g0 + g1

        gxs = []
        for k in range(EXP_PER_DEV):
            ge = EXP_PER_DEV * my + k
            sel = (idx0 == ge) | (idx1 == ge)
            p_e = jnp.sum(probs * (e_ids == ge), axis=1, keepdims=True)
            gate = jnp.where(sel, p_e / gs, 0.0)
            gxs.append((gate * xv).astype(jnp.bfloat16))
        ew_cp.wait()
        w_b = [ew_vmem[k].astype(jnp.bfloat16) for k in range(EXP_PER_DEV)]

        CK = N_TOK // N_CHUNK
        H = N_CHUNK // 2
        order = [h * H + s for s in range(H) for h in range(2)]

        def mk(stage, c, p):
            sl = pl.ds(c * CK, CK)
            return pltpu.make_async_remote_copy(
                src_ref=send_buf.at[stage, sl],
                dst_ref=recv_buf.at[stage, sl],
                send_sem=send_sems.at[stage, c],
                recv_sem=recv_sems.at[stage, c],
                device_id=(p,),
                device_id_type=pl.DeviceIdType.MESH,
            )

        s1 = [mk(0, c, p1 if c < H else p2) for c in range(N_CHUNK)]
        s2 = [mk(1, c, p2 if c < H else p1) for c in range(N_CHUNK)]

        partials = [None] * N_CHUNK
        for i, c in enumerate(order):
            r = pl.ds(c * CK, CK)
            pc = jnp.dot(gxs[0][c * CK:(c + 1) * CK, :], w_b[0],
                         preferred_element_type=jnp.float32)
            for k in range(1, EXP_PER_DEV):
                pc = pc + jnp.dot(gxs[k][c * CK:(c + 1) * CK, :], w_b[k],
                                  preferred_element_type=jnp.float32)
            partials[c] = pc
            send_buf[0, r, :] = pc.astype(jnp.bfloat16)
            if i == 0:
                pl.semaphore_wait(barrier_sem, 2)
            s1[c].start()

        accs = [None] * N_CHUNK
        for c in order:
            s1[c].wait()
            sl = pl.ds(c * CK, CK)
            a = partials[c] + recv_buf[0, sl, :].astype(jnp.float32)
            accs[c] = a
            send_buf[1, sl, :] = a.astype(jnp.bfloat16)
            s2[c].start()

        for c in order:
            s2[c].wait()
            sl = pl.ds(c * CK, CK)
            out_ref[sl, :] = (accs[c] + recv_buf[1, sl, :].astype(jnp.float32)
                              ).astype(jnp.bfloat16)

    return pl.pallas_call(
        body,
        out_shape=jax.ShapeDtypeStruct((N_TOK, D_OUT), jnp.bfloat16),
        in_specs=[
            pl.BlockSpec(memory_space=pltpu.VMEM),
            pl.BlockSpec(memory_space=pltpu.VMEM),
            pl.BlockSpec(memory_space=pltpu.VMEM),
            pl.BlockSpec(memory_space=pl.ANY),
        ],
        out_specs=pl.BlockSpec(memory_space=pltpu.VMEM),
        scratch_shapes=[
            pltpu.VMEM((2, N_TOK, D_OUT), jnp.bfloat16),
            pltpu.VMEM((2, N_TOK, D_OUT), jnp.bfloat16),
            pltpu.VMEM((EXP_PER_DEV, D_IN, D_OUT), jnp.float32),
            pltpu.SemaphoreType.DMA,
            pltpu.SemaphoreType.DMA((2, N_CHUNK)),
            pltpu.SemaphoreType.DMA((2, N_CHUNK)),
        ],
        compiler_params=pltpu.CompilerParams(collective_id=0),
    )(x, router_W, route_idx, expert_W)
